# baseline (speedup 1.0000x reference)
import functools

import jax
import jax.numpy as jnp
import numpy as np
from jax import lax
from jax.experimental import pallas as pl
from jax.experimental.pallas import tpu as pltpu
from jax.experimental.pallas import tpu_sc as plsc

NUM_ENT = 10000
IN_C = 128
OUT_C = 128
E_HALF = 320000

NC = 2
NS = 16
NW = NC * NS
NPAD = 10240
ROWS = NPAD // NS

CHUNK = 24
NCHUNK = 417
EPW = NCHUNK * CHUNK
E_PAD = NW * EPW
CHUNK_D = 72
NCHUNK_D = EPW // CHUNK_D

_mesh = plsc.VectorSubcoreMesh(core_axis_name="c", subcore_axis_name="s")

_Q = np.zeros(IN_C, np.int32)
for _j in range(IN_C // 32):
    for _i in range(16):
        _Q[32 * _j + 2 * _i] = 32 * _j + _i
        _Q[32 * _j + 2 * _i + 1] = 32 * _j + 16 + _i


def _gelu(x):
    return 0.5 * x * (1 + jnp.tanh(0.7978845608 * (x + 0.044715 * x ** 3)))


@functools.partial(
    pl.kernel,
    mesh=_mesh,
    out_type=jax.ShapeDtypeStruct((NC, NPAD, OUT_C), jnp.float32),
    scratch_types=[
        pltpu.VMEM((1, CHUNK_D), jnp.int32),
        pltpu.VMEM((1, CHUNK_D), jnp.int32),
        pltpu.VMEM((CHUNK_D, OUT_C), jnp.float32),
        pltpu.VMEM_SHARED((NPAD, OUT_C), jnp.float32),
        pltpu.SemaphoreType.DMA,
        pltpu.SemaphoreType.DMA,
    ],
)
def _deg_kernel(dst_hbm, ones_hbm, z_hbm, out_hbm,
                idx0, idx1, ones_v, deg_sh, d0, d1):
    cid = lax.axis_index("c")
    sid = lax.axis_index("s")
    wid = sid * NC + cid
    pltpu.sync_copy(ones_hbm, ones_v)
    pltpu.sync_copy(z_hbm, deg_sh.at[pl.ds(sid * ROWS, ROWS)])
    plsc.subcore_barrier()

    pltpu.async_copy(dst_hbm.at[wid, pl.ds(0, 1)], idx0, d0)
    pltpu.async_copy(dst_hbm.at[wid, pl.ds(1, 1)], idx1, d1)

    def pair(t, carry):
        c0 = 2 * t
        c1 = c0 + 1
        pltpu.make_async_copy(dst_hbm.at[wid, pl.ds(c0, 1)], idx0, d0).wait()
        pltpu.sync_copy(ones_v, deg_sh.at[idx0.at[0]], add=True)
        pltpu.async_copy(dst_hbm.at[wid, pl.ds(c0 + 2, 1)], idx0, d0)
        pltpu.make_async_copy(dst_hbm.at[wid, pl.ds(c1, 1)], idx1, d1).wait()
        pltpu.sync_copy(ones_v, deg_sh.at[idx1.at[0]], add=True)
        nxt = jnp.minimum(c1 + 2, NCHUNK_D - 2)
        pltpu.async_copy(dst_hbm.at[wid, pl.ds(nxt, 1)], idx1, d1)
        return carry

    lax.fori_loop(0, (NCHUNK_D - 1) // 2, pair, 0)
    pltpu.make_async_copy(dst_hbm.at[wid, pl.ds(NCHUNK_D - 1, 1)], idx0,
                          d0).wait()
    pltpu.sync_copy(ones_v, deg_sh.at[idx0.at[0]], add=True)
    pltpu.make_async_copy(dst_hbm.at[wid, pl.ds(0, 1)], idx1, d1).wait()
    plsc.subcore_barrier()
    pltpu.sync_copy(deg_sh.at[pl.ds(sid * ROWS, ROWS)],
                    out_hbm.at[cid, pl.ds(sid * ROWS, ROWS)])


@functools.partial(
    pl.kernel,
    mesh=_mesh,
    out_type=jax.ShapeDtypeStruct((NC, NPAD, OUT_C), jnp.float32),
    scratch_types=[
        pltpu.VMEM((4, CHUNK), jnp.int32),
        pltpu.VMEM((4, CHUNK), jnp.int32),
        pltpu.VMEM((CHUNK, IN_C), jnp.int32),
        pltpu.VMEM((CHUNK, IN_C), jnp.int32),
        pltpu.VMEM((CHUNK, IN_C), jnp.int32),
        pltpu.VMEM((CHUNK, IN_C), jnp.int32),
        pltpu.VMEM((CHUNK, IN_C), jnp.int32),
        pltpu.VMEM((CHUNK, IN_C), jnp.int32),
        pltpu.VMEM((CHUNK, OUT_C), jnp.float32),
        pltpu.VMEM_SHARED((NPAD, OUT_C), jnp.float32),
        pltpu.SemaphoreType.DMA,
        pltpu.SemaphoreType.DMA,
        pltpu.SemaphoreType.DMA,
        pltpu.SemaphoreType.DMA,
    ],
)
def _edge_kernel(tn_hbm, rwt_hbm, idx_hbm, z_hbm, out_hbm,
                 idx0, idx1, xg0, xg1, rwg0, rwg1, ewg0, ewg1, vout, acc_sh,
                 i0, i1, g0, g1):
    cid = lax.axis_index("c")
    sid = lax.axis_index("s")
    wid = sid * NC + cid
    pltpu.sync_copy(z_hbm, acc_sh.at[pl.ds(sid * ROWS, ROWS)])
    plsc.subcore_barrier()

    sets = (
        (idx0, xg0, rwg0, ewg0, g0, i0),
        (idx1, xg1, rwg1, ewg1, g1, i1),
    )

    def fire_gathers(s):
        idx, xg, rwg, ewg, g, _ = sets[s]
        pltpu.async_copy(tn_hbm.at[idx.at[0]], xg, g)
        pltpu.async_copy(rwt_hbm.at[idx.at[1]], rwg, g)
        pltpu.async_copy(tn_hbm.at[idx.at[2]], ewg, g)

    def wait_gathers(s):
        idx, xg, rwg, ewg, g, _ = sets[s]
        pltpu.make_async_copy(tn_hbm.at[idx.at[0]], xg, g).wait()
        pltpu.make_async_copy(rwt_hbm.at[idx.at[1]], rwg, g).wait()
        pltpu.make_async_copy(tn_hbm.at[idx.at[2]], ewg, g).wait()

    def fire_idx(s, c):
        idx, _, _, _, _, sem = sets[s]
        pltpu.async_copy(idx_hbm.at[wid, c], idx, sem)

    def wait_idx(s, c):
        idx, _, _, _, _, sem = sets[s]
        pltpu.make_async_copy(idx_hbm.at[wid, c], idx, sem).wait()

    hi_mask = jnp.int32(-65536)

    def split_bf16(w):
        a = lax.bitcast_convert_type(lax.shift_left(w, 16), jnp.float32)
        b = lax.bitcast_convert_type(lax.bitwise_and(w, hi_mask), jnp.float32)
        return a, b

    def compute_scatter(s):
        idx, xg, rwg, ewg, _, _ = sets[s]

        def edge_body(e, carry):
            for j in range(IN_C // 32):
                ds16 = pl.ds(j * 16, 16)
                dshi = pl.ds(IN_C // 2 + j * 16, 16)
                xa, xb = split_bf16(xg[e, ds16])
                ea, eb = split_bf16(ewg[e, dshi])
                ra, rb = split_bf16(rwg[e, ds16])
                wa, wb = split_bf16(rwg[e, dshi])
                vout[e, pl.ds(j * 32, 16)] = xa * (ra * ea + wa)
                vout[e, pl.ds(j * 32 + 16, 16)] = xb * (rb * eb + wb)
            return carry

        lax.fori_loop(0, CHUNK, edge_body, 0)
        pltpu.sync_copy(vout, acc_sh.at[idx.at[3]], add=True)

    fire_idx(0, 0)
    fire_idx(1, 1)
    wait_idx(0, 0)
    fire_gathers(0)

    def pair(t, carry):
        c0 = 2 * t
        c1 = c0 + 1
        wait_gathers(0)
        wait_idx(1, c1)
        fire_gathers(1)
        compute_scatter(0)
        fire_idx(0, c0 + 2)
        wait_gathers(1)
        wait_idx(0, c0 + 2)
        fire_gathers(0)
        compute_scatter(1)
        fire_idx(1, jnp.minimum(c1 + 2, NCHUNK - 1))
        return carry

    lax.fori_loop(0, NCHUNK // 2, pair, 0)
    wait_gathers(0)
    compute_scatter(0)
    wait_idx(1, NCHUNK - 1)

    plsc.subcore_barrier()
    pltpu.sync_copy(acc_sh.at[pl.ds(sid * ROWS, ROWS)],
                    out_hbm.at[cid, pl.ds(sid * ROWS, ROWS)])


def _prep_body(degp, xp, ew2p, lr, w1l, x2, sl, dinv):
    deg = degp[0, :, 0:1] + degp[1, :, 0:1]
    di = jnp.where(deg > 0, lax.rsqrt(deg), 0.0)
    dinv[...] = di
    x2[...] = xp[...] * di
    t = xp[...] * ew2p[...] * lr[...]
    sl[...] = _gelu(jnp.dot(t, w1l[...], preferred_element_type=jnp.float32))


def _final_body(acc, dinv, sl, w1o, attn, gam, bet, out):
    acc2 = acc[0] + acc[1]
    aggr = jnp.dot(acc2, w1o[...], preferred_element_type=jnp.float32) * dinv[...]
    res1 = _gelu(aggr)
    a = jnp.sum(res1 * attn[...], axis=1, keepdims=True)
    out2 = a * res1
    scale = gam[...] * (1.0 / jnp.sqrt(1.0 + 1e-5))
    out[...] = (0.75 * sl[...] + 0.25 * out2) * scale + bet[...]


def _rel_body(re, wr, out):
    out[...] = jnp.dot(re[...], wr[...], preferred_element_type=jnp.float32)


def kernel(x, rel_embed, edge_index, edge_type, w1_loop, w1_out, w_rel,
           loop_rel, attn_w, rel_weight1, ent_weight1, ent_weight2,
           bn_gamma, bn_beta):
    ei = edge_index.astype(jnp.int32)
    et = edge_type.astype(jnp.int32)
    npad_e = E_PAD - E_HALF
    pad_dst = (NUM_ENT + (jnp.arange(npad_e) % (NPAD - NUM_ENT))).astype(
        jnp.int32)
    zpad = jnp.zeros((npad_e,), jnp.int32)
    dst = jnp.concatenate([ei[0, E_HALF:], pad_dst])
    src = jnp.concatenate([ei[1, E_HALF:], zpad])
    in0 = jnp.concatenate([ei[0, :E_HALF], zpad])
    typ = jnp.concatenate([et[E_HALF:], zpad])
    idx4 = jnp.stack([
        src.reshape(NW, NCHUNK, CHUNK),
        typ.reshape(NW, NCHUNK, CHUNK),
        in0.reshape(NW, NCHUNK, CHUNK),
        dst.reshape(NW, NCHUNK, CHUNK),
    ], axis=2)
    dstd = dst.reshape(NW, NCHUNK_D, CHUNK_D)

    pad = ((0, NPAD - NUM_ENT), (0, 0))
    xp = jnp.pad(x, pad)
    ew2p = jnp.pad(ent_weight2, pad)

    def _to_i32(a):
        n, m = a.shape
        return lax.bitcast_convert_type(a.reshape(n, m // 2, 2), jnp.int32)

    nrel = rel_embed.shape[0]
    rwt = _to_i32(jnp.concatenate(
        [rel_embed[:, _Q], rel_weight1[:nrel][:, _Q]], axis=1
    ).astype(jnp.bfloat16))
    ew1p = jnp.pad(ent_weight1, pad)

    ones128 = jnp.ones((CHUNK_D, OUT_C), jnp.float32)
    z128 = jnp.zeros((ROWS, OUT_C), jnp.float32)

    degp = _deg_kernel(dstd, ones128, z128)

    x2, sl, dinv = pl.pallas_call(
        _prep_body,
        out_shape=[
            jax.ShapeDtypeStruct((NPAD, IN_C), jnp.float32),
            jax.ShapeDtypeStruct((NPAD, OUT_C), jnp.float32),
            jax.ShapeDtypeStruct((NPAD, 1), jnp.float32),
        ],
    )(degp, xp, ew2p, loop_rel, w1_loop)

    tn = _to_i32(jnp.concatenate(
        [x2[:, _Q], ew1p[:, _Q]], axis=1).astype(jnp.bfloat16))
    acc = _edge_kernel(tn, rwt, idx4, z128)

    out = pl.pallas_call(
        _final_body,
        out_shape=jax.ShapeDtypeStruct((NPAD, OUT_C), jnp.float32),
    )(acc, dinv, sl, w1_out, attn_w, bn_gamma.reshape(1, OUT_C),
      bn_beta.reshape(1, OUT_C))

    rel1 = pl.pallas_call(
        _rel_body,
        out_shape=jax.ShapeDtypeStruct((rel_embed.shape[0], OUT_C), jnp.float32),
    )(rel_embed, w_rel)

    return out[:NUM_ENT], rel1


# speedup vs baseline: 1.1061x; 1.1061x over previous
import functools

import jax
import jax.numpy as jnp
import numpy as np
from jax import lax
from jax.experimental import pallas as pl
from jax.experimental.pallas import tpu as pltpu
from jax.experimental.pallas import tpu_sc as plsc

NUM_ENT = 10000
IN_C = 128
OUT_C = 128
E_HALF = 320000

NC = 2
NS = 16
NW = NC * NS
NPAD = 10240
ROWS = NPAD // NS

CHUNK = 24
NCHUNK = 417
EPW = NCHUNK * CHUNK
E_PAD = NW * EPW
CHUNK_D = 72
NCHUNK_D = EPW // CHUNK_D

_mesh = plsc.VectorSubcoreMesh(core_axis_name="c", subcore_axis_name="s")

_Q = np.zeros(IN_C, np.int32)
for _j in range(IN_C // 32):
    for _i in range(16):
        _Q[32 * _j + 2 * _i] = 32 * _j + _i
        _Q[32 * _j + 2 * _i + 1] = 32 * _j + 16 + _i


def _gelu(x):
    return 0.5 * x * (1 + jnp.tanh(0.7978845608 * (x + 0.044715 * x ** 3)))


@functools.partial(
    pl.kernel,
    mesh=_mesh,
    out_type=jax.ShapeDtypeStruct((NC, NPAD, OUT_C), jnp.float32),
    scratch_types=[
        pltpu.VMEM((1, CHUNK_D), jnp.int32),
        pltpu.VMEM((1, CHUNK_D), jnp.int32),
        pltpu.VMEM((CHUNK_D, OUT_C), jnp.float32),
        pltpu.VMEM_SHARED((NPAD, OUT_C), jnp.float32),
        pltpu.SemaphoreType.DMA,
        pltpu.SemaphoreType.DMA,
    ],
)
def _deg_kernel(dst_hbm, ones_hbm, z_hbm, out_hbm,
                idx0, idx1, ones_v, deg_sh, d0, d1):
    cid = lax.axis_index("c")
    sid = lax.axis_index("s")
    wid = sid * NC + cid
    pltpu.sync_copy(ones_hbm, ones_v)
    pltpu.sync_copy(z_hbm, deg_sh.at[pl.ds(sid * ROWS, ROWS)])
    plsc.subcore_barrier()

    pltpu.async_copy(dst_hbm.at[wid, pl.ds(0, 1)], idx0, d0)
    pltpu.async_copy(dst_hbm.at[wid, pl.ds(1, 1)], idx1, d1)

    def pair(t, carry):
        c0 = 2 * t
        c1 = c0 + 1
        pltpu.make_async_copy(dst_hbm.at[wid, pl.ds(c0, 1)], idx0, d0).wait()
        pltpu.sync_copy(ones_v, deg_sh.at[idx0.at[0]], add=True)
        pltpu.async_copy(dst_hbm.at[wid, pl.ds(c0 + 2, 1)], idx0, d0)
        pltpu.make_async_copy(dst_hbm.at[wid, pl.ds(c1, 1)], idx1, d1).wait()
        pltpu.sync_copy(ones_v, deg_sh.at[idx1.at[0]], add=True)
        nxt = jnp.minimum(c1 + 2, NCHUNK_D - 2)
        pltpu.async_copy(dst_hbm.at[wid, pl.ds(nxt, 1)], idx1, d1)
        return carry

    lax.fori_loop(0, (NCHUNK_D - 1) // 2, pair, 0)
    pltpu.make_async_copy(dst_hbm.at[wid, pl.ds(NCHUNK_D - 1, 1)], idx0,
                          d0).wait()
    pltpu.sync_copy(ones_v, deg_sh.at[idx0.at[0]], add=True)
    pltpu.make_async_copy(dst_hbm.at[wid, pl.ds(0, 1)], idx1, d1).wait()
    plsc.subcore_barrier()
    pltpu.sync_copy(deg_sh.at[pl.ds(sid * ROWS, ROWS)],
                    out_hbm.at[cid, pl.ds(sid * ROWS, ROWS)])


@functools.partial(
    pl.kernel,
    mesh=_mesh,
    out_type=jax.ShapeDtypeStruct((NC, NPAD, OUT_C), jnp.float32),
    scratch_types=[
        pltpu.VMEM((4, CHUNK), jnp.int32),
        pltpu.VMEM((4, CHUNK), jnp.int32),
        pltpu.VMEM((CHUNK, IN_C), jnp.float32),
        pltpu.VMEM((CHUNK, IN_C), jnp.float32),
        pltpu.VMEM((CHUNK, IN_C), jnp.int32),
        pltpu.VMEM((CHUNK, IN_C), jnp.int32),
        pltpu.VMEM((CHUNK, IN_C), jnp.float32),
        pltpu.VMEM((CHUNK, IN_C), jnp.float32),
        pltpu.VMEM_SHARED((NPAD, OUT_C), jnp.float32),
        pltpu.SemaphoreType.DMA,
        pltpu.SemaphoreType.DMA,
        pltpu.SemaphoreType.DMA,
        pltpu.SemaphoreType.DMA,
        pltpu.SemaphoreType.DMA,
        pltpu.SemaphoreType.DMA,
    ],
)
def _edge_kernel(x2_hbm, rwt_hbm, ew1_hbm, idx_hbm, z_hbm, out_hbm,
                 idx0, idx1, xg0, xg1, rwg0, rwg1, ewg0, ewg1, acc_sh,
                 i0, i1, g0, g1, sc0, sc1):
    cid = lax.axis_index("c")
    sid = lax.axis_index("s")
    wid = sid * NC + cid
    pltpu.sync_copy(z_hbm, acc_sh.at[pl.ds(sid * ROWS, ROWS)])
    plsc.subcore_barrier()

    sets = (
        (idx0, xg0, rwg0, ewg0, g0, i0, sc0),
        (idx1, xg1, rwg1, ewg1, g1, i1, sc1),
    )

    def fire_gathers(s):
        idx, xg, rwg, ewg, g, _, _ = sets[s]
        pltpu.async_copy(x2_hbm.at[idx.at[0]], xg, g)
        pltpu.async_copy(rwt_hbm.at[idx.at[1]], rwg, g)
        pltpu.async_copy(ew1_hbm.at[idx.at[2]], ewg, g)

    def wait_gathers(s):
        idx, xg, rwg, ewg, g, _, _ = sets[s]
        pltpu.make_async_copy(x2_hbm.at[idx.at[0]], xg, g).wait()
        pltpu.make_async_copy(rwt_hbm.at[idx.at[1]], rwg, g).wait()
        pltpu.make_async_copy(ew1_hbm.at[idx.at[2]], ewg, g).wait()

    def fire_idx(s, c):
        idx, _, _, _, _, sem, _ = sets[s]
        pltpu.async_copy(idx_hbm.at[wid, c], idx, sem)

    def wait_idx(s, c):
        idx, _, _, _, _, sem, _ = sets[s]
        pltpu.make_async_copy(idx_hbm.at[wid, c], idx, sem).wait()

    def wait_scatter(s):
        idx, xg, _, _, _, _, ssem = sets[s]
        pltpu.make_async_copy(xg, acc_sh.at[idx.at[3]], ssem).wait()

    hi_mask = jnp.int32(-65536)

    def split_bf16(w):
        a = lax.bitcast_convert_type(lax.shift_left(w, 16), jnp.float32)
        b = lax.bitcast_convert_type(lax.bitwise_and(w, hi_mask), jnp.float32)
        return a, b

    def compute_scatter(s):
        idx, xg, rwg, ewg, _, _, ssem = sets[s]

        def edge_pair(p, carry):
            for u in range(2):
                e = 2 * p + u
                for j in range(IN_C // 32):
                    dsa = pl.ds(j * 32, 16)
                    dsb = pl.ds(j * 32 + 16, 16)
                    ra, rb = split_bf16(rwg[e, pl.ds(j * 16, 16)])
                    wa, wb = split_bf16(rwg[e, pl.ds(IN_C // 2 + j * 16, 16)])
                    xg[e, dsa] = xg[e, dsa] * (ra * ewg[e, dsa] + wa)
                    xg[e, dsb] = xg[e, dsb] * (rb * ewg[e, dsb] + wb)
            return carry

        lax.fori_loop(0, CHUNK // 2, edge_pair, 0)
        pltpu.async_copy(xg, acc_sh.at[idx.at[3]], ssem, add=True)

    fire_idx(0, 0)
    fire_idx(1, 1)
    pltpu.async_copy(z_hbm.at[pl.ds(0, CHUNK)], xg1, sc1)
    wait_idx(0, 0)
    fire_gathers(0)

    def pair(t, carry):
        c0 = 2 * t
        c1 = c0 + 1
        wait_gathers(0)
        wait_idx(1, c1)
        wait_scatter(1)
        fire_gathers(1)
        compute_scatter(0)
        fire_idx(0, c0 + 2)
        wait_gathers(1)
        wait_idx(0, c0 + 2)
        wait_scatter(0)
        fire_gathers(0)
        compute_scatter(1)
        fire_idx(1, jnp.minimum(c1 + 2, NCHUNK - 1))
        return carry

    lax.fori_loop(0, NCHUNK // 2, pair, 0)
    wait_gathers(0)
    compute_scatter(0)
    wait_scatter(0)
    wait_scatter(1)
    wait_idx(1, NCHUNK - 1)

    plsc.subcore_barrier()
    pltpu.sync_copy(acc_sh.at[pl.ds(sid * ROWS, ROWS)],
                    out_hbm.at[cid, pl.ds(sid * ROWS, ROWS)])


def _prep_body(degp, xp, ew2p, lr, w1l, x2, sl, dinv):
    deg = degp[0, :, 0:1] + degp[1, :, 0:1]
    di = jnp.where(deg > 0, lax.rsqrt(deg), 0.0)
    dinv[...] = di
    x2[...] = xp[...] * di
    t = xp[...] * ew2p[...] * lr[...]
    sl[...] = _gelu(jnp.dot(t, w1l[...], preferred_element_type=jnp.float32))


def _final_body(acc, dinv, sl, w1o, attn, gam, bet, out):
    acc2 = acc[0] + acc[1]
    aggr = jnp.dot(acc2, w1o[...], preferred_element_type=jnp.float32) * dinv[...]
    res1 = _gelu(aggr)
    a = jnp.sum(res1 * attn[...], axis=1, keepdims=True)
    out2 = a * res1
    scale = gam[...] * (1.0 / jnp.sqrt(1.0 + 1e-5))
    out[...] = (0.75 * sl[...] + 0.25 * out2) * scale + bet[...]


def _rel_body(re, wr, out):
    out[...] = jnp.dot(re[...], wr[...], preferred_element_type=jnp.float32)


def kernel(x, rel_embed, edge_index, edge_type, w1_loop, w1_out, w_rel,
           loop_rel, attn_w, rel_weight1, ent_weight1, ent_weight2,
           bn_gamma, bn_beta):
    ei = edge_index.astype(jnp.int32)
    et = edge_type.astype(jnp.int32)
    npad_e = E_PAD - E_HALF
    pad_dst = (NUM_ENT + (jnp.arange(npad_e) % (NPAD - NUM_ENT))).astype(
        jnp.int32)
    zpad = jnp.zeros((npad_e,), jnp.int32)
    dst = jnp.concatenate([ei[0, E_HALF:], pad_dst])
    src = jnp.concatenate([ei[1, E_HALF:], zpad])
    in0 = jnp.concatenate([ei[0, :E_HALF], zpad])
    typ = jnp.concatenate([et[E_HALF:], zpad])
    idx4 = jnp.stack([
        src.reshape(NW, NCHUNK, CHUNK),
        typ.reshape(NW, NCHUNK, CHUNK),
        in0.reshape(NW, NCHUNK, CHUNK),
        dst.reshape(NW, NCHUNK, CHUNK),
    ], axis=2)
    dstd = dst.reshape(NW, NCHUNK_D, CHUNK_D)

    pad = ((0, NPAD - NUM_ENT), (0, 0))
    xp = jnp.pad(x, pad)
    ew2p = jnp.pad(ent_weight2, pad)

    def _to_i32(a):
        n, m = a.shape
        return lax.bitcast_convert_type(a.reshape(n, m // 2, 2), jnp.int32)

    nrel = rel_embed.shape[0]
    rwt = _to_i32(jnp.concatenate(
        [rel_embed[:, _Q], rel_weight1[:nrel][:, _Q]], axis=1
    ).astype(jnp.bfloat16))

    ones128 = jnp.ones((CHUNK_D, OUT_C), jnp.float32)
    z128 = jnp.zeros((ROWS, OUT_C), jnp.float32)

    degp = _deg_kernel(dstd, ones128, z128)

    x2, sl, dinv = pl.pallas_call(
        _prep_body,
        out_shape=[
            jax.ShapeDtypeStruct((NPAD, IN_C), jnp.float32),
            jax.ShapeDtypeStruct((NPAD, OUT_C), jnp.float32),
            jax.ShapeDtypeStruct((NPAD, 1), jnp.float32),
        ],
    )(degp, xp, ew2p, loop_rel, w1_loop)

    acc = _edge_kernel(x2, rwt, ent_weight1, idx4, z128)

    out = pl.pallas_call(
        _final_body,
        out_shape=jax.ShapeDtypeStruct((NPAD, OUT_C), jnp.float32),
    )(acc, dinv, sl, w1_out, attn_w, bn_gamma.reshape(1, OUT_C),
      bn_beta.reshape(1, OUT_C))

    rel1 = pl.pallas_call(
        _rel_body,
        out_shape=jax.ShapeDtypeStruct((rel_embed.shape[0], OUT_C), jnp.float32),
    )(rel_embed, w_rel)

    return out[:NUM_ENT], rel1


# speedup vs baseline: 1.1128x; 1.0061x over previous
import functools

import jax
import jax.numpy as jnp
import numpy as np
from jax import lax
from jax.experimental import pallas as pl
from jax.experimental.pallas import tpu as pltpu
from jax.experimental.pallas import tpu_sc as plsc

NUM_ENT = 10000
IN_C = 128
OUT_C = 128
E_HALF = 320000

NC = 2
NS = 16
NW = NC * NS
NPAD = 10240
ROWS = NPAD // NS

CHUNK = 24
NCHUNK = 417
EPW = NCHUNK * CHUNK
E_PAD = NW * EPW
CHUNK_D = 72
NCHUNK_D = EPW // CHUNK_D

_mesh = plsc.VectorSubcoreMesh(core_axis_name="c", subcore_axis_name="s")

_Q = np.zeros(IN_C, np.int32)
for _j in range(IN_C // 32):
    for _i in range(16):
        _Q[32 * _j + 2 * _i] = 32 * _j + _i
        _Q[32 * _j + 2 * _i + 1] = 32 * _j + 16 + _i


def _gelu(x):
    return 0.5 * x * (1 + jnp.tanh(0.7978845608 * (x + 0.044715 * x ** 3)))


@functools.partial(
    pl.kernel,
    mesh=_mesh,
    out_type=jax.ShapeDtypeStruct((NC, NPAD, OUT_C), jnp.float32),
    scratch_types=[
        pltpu.VMEM((1, CHUNK_D), jnp.int32),
        pltpu.VMEM((1, CHUNK_D), jnp.int32),
        pltpu.VMEM((CHUNK_D, OUT_C), jnp.float32),
        pltpu.VMEM_SHARED((NPAD, OUT_C), jnp.float32),
        pltpu.SemaphoreType.DMA,
        pltpu.SemaphoreType.DMA,
    ],
)
def _deg_kernel(dst_hbm, ones_hbm, z_hbm, out_hbm,
                idx0, idx1, ones_v, deg_sh, d0, d1):
    cid = lax.axis_index("c")
    sid = lax.axis_index("s")
    wid = sid * NC + cid
    pltpu.sync_copy(ones_hbm, ones_v)
    pltpu.sync_copy(z_hbm, deg_sh.at[pl.ds(sid * ROWS, ROWS)])
    plsc.subcore_barrier()

    pltpu.async_copy(dst_hbm.at[wid, pl.ds(0, 1)], idx0, d0)
    pltpu.async_copy(dst_hbm.at[wid, pl.ds(1, 1)], idx1, d1)

    def pair(t, carry):
        c0 = 2 * t
        c1 = c0 + 1
        pltpu.make_async_copy(dst_hbm.at[wid, pl.ds(c0, 1)], idx0, d0).wait()
        pltpu.sync_copy(ones_v, deg_sh.at[idx0.at[0]], add=True)
        pltpu.async_copy(dst_hbm.at[wid, pl.ds(c0 + 2, 1)], idx0, d0)
        pltpu.make_async_copy(dst_hbm.at[wid, pl.ds(c1, 1)], idx1, d1).wait()
        pltpu.sync_copy(ones_v, deg_sh.at[idx1.at[0]], add=True)
        nxt = jnp.minimum(c1 + 2, NCHUNK_D - 2)
        pltpu.async_copy(dst_hbm.at[wid, pl.ds(nxt, 1)], idx1, d1)
        return carry

    lax.fori_loop(0, (NCHUNK_D - 1) // 2, pair, 0)
    pltpu.make_async_copy(dst_hbm.at[wid, pl.ds(NCHUNK_D - 1, 1)], idx0,
                          d0).wait()
    pltpu.sync_copy(ones_v, deg_sh.at[idx0.at[0]], add=True)
    pltpu.make_async_copy(dst_hbm.at[wid, pl.ds(0, 1)], idx1, d1).wait()
    plsc.subcore_barrier()
    pltpu.sync_copy(deg_sh.at[pl.ds(sid * ROWS, ROWS)],
                    out_hbm.at[cid, pl.ds(sid * ROWS, ROWS)])


@functools.partial(
    pl.kernel,
    mesh=_mesh,
    out_type=jax.ShapeDtypeStruct((NC, NPAD, OUT_C), jnp.float32),
    scratch_types=[
        pltpu.VMEM((4, CHUNK), jnp.int32),
        pltpu.VMEM((4, CHUNK), jnp.int32),
        pltpu.VMEM((CHUNK, IN_C), jnp.float32),
        pltpu.VMEM((CHUNK, IN_C), jnp.float32),
        pltpu.VMEM((CHUNK, IN_C), jnp.int32),
        pltpu.VMEM((CHUNK, IN_C), jnp.int32),
        pltpu.VMEM((CHUNK, IN_C), jnp.float32),
        pltpu.VMEM((CHUNK, IN_C), jnp.float32),
        pltpu.VMEM_SHARED((NPAD, OUT_C), jnp.float32),
        pltpu.SemaphoreType.DMA,
        pltpu.SemaphoreType.DMA,
        pltpu.SemaphoreType.DMA,
        pltpu.SemaphoreType.DMA,
        pltpu.SemaphoreType.DMA,
        pltpu.SemaphoreType.DMA,
    ],
)
def _edge_kernel(x2_hbm, rwt_hbm, ew1_hbm, idx_hbm, z_hbm, out_hbm,
                 idx0, idx1, xg0, xg1, rwg0, rwg1, ewg0, ewg1, acc_sh,
                 i0, i1, g0, g1, sc0, sc1):
    cid = lax.axis_index("c")
    sid = lax.axis_index("s")
    wid = sid * NC + cid
    pltpu.sync_copy(z_hbm, acc_sh.at[pl.ds(sid * ROWS, ROWS)])
    plsc.subcore_barrier()

    sets = (
        (idx0, xg0, rwg0, ewg0, g0, i0, sc0),
        (idx1, xg1, rwg1, ewg1, g1, i1, sc1),
    )

    def fire_gathers(s):
        idx, xg, rwg, ewg, g, _, _ = sets[s]
        pltpu.async_copy(x2_hbm.at[idx.at[0]], xg, g)
        pltpu.async_copy(rwt_hbm.at[idx.at[1]], rwg, g)
        pltpu.async_copy(ew1_hbm.at[idx.at[2]], ewg, g)

    def wait_gathers(s):
        idx, xg, rwg, ewg, g, _, _ = sets[s]
        pltpu.make_async_copy(x2_hbm.at[idx.at[0]], xg, g).wait()
        pltpu.make_async_copy(rwt_hbm.at[idx.at[1]], rwg, g).wait()
        pltpu.make_async_copy(ew1_hbm.at[idx.at[2]], ewg, g).wait()

    def fire_idx(s, c):
        idx, _, _, _, _, sem, _ = sets[s]
        pltpu.async_copy(idx_hbm.at[wid, c], idx, sem)

    def wait_idx(s, c):
        idx, _, _, _, _, sem, _ = sets[s]
        pltpu.make_async_copy(idx_hbm.at[wid, c], idx, sem).wait()

    def wait_scatter(s):
        idx, xg, _, _, _, _, ssem = sets[s]
        pltpu.make_async_copy(xg, acc_sh.at[idx.at[3]], ssem).wait()

    hi_mask = jnp.int32(-65536)

    def split_bf16(w):
        a = lax.bitcast_convert_type(lax.shift_left(w, 16), jnp.float32)
        b = lax.bitcast_convert_type(lax.bitwise_and(w, hi_mask), jnp.float32)
        return a, b

    def compute_scatter(s):
        idx, xg, rwg, ewg, _, _, ssem = sets[s]

        def edge_pair(p, carry):
            for u in range(2):
                e = 2 * p + u
                for j in range(IN_C // 32):
                    dsa = pl.ds(j * 32, 16)
                    dsb = pl.ds(j * 32 + 16, 16)
                    ra, rb = split_bf16(rwg[e, pl.ds(j * 16, 16)])
                    wa, wb = split_bf16(rwg[e, pl.ds(IN_C // 2 + j * 16, 16)])
                    xg[e, dsa] = xg[e, dsa] * (ra * ewg[e, dsa] + wa)
                    xg[e, dsb] = xg[e, dsb] * (rb * ewg[e, dsb] + wb)
            return carry

        lax.fori_loop(0, CHUNK // 2, edge_pair, 0)
        pltpu.async_copy(xg, acc_sh.at[idx.at[3]], ssem, add=True)

    fire_idx(0, 0)
    fire_idx(1, 1)
    pltpu.async_copy(z_hbm.at[pl.ds(0, CHUNK)], xg1, sc1)
    wait_idx(0, 0)
    fire_gathers(0)

    def pair(t, carry):
        c0 = 2 * t
        c1 = c0 + 1
        wait_gathers(0)
        wait_idx(1, c1)
        wait_scatter(1)
        fire_gathers(1)
        compute_scatter(0)
        fire_idx(0, c0 + 2)
        wait_gathers(1)
        wait_idx(0, c0 + 2)
        wait_scatter(0)
        fire_gathers(0)
        compute_scatter(1)
        fire_idx(1, jnp.minimum(c1 + 2, NCHUNK - 1))
        return carry

    lax.fori_loop(0, NCHUNK // 2, pair, 0)
    wait_gathers(0)
    compute_scatter(0)
    wait_scatter(0)
    wait_scatter(1)
    wait_idx(1, NCHUNK - 1)

    plsc.subcore_barrier()
    pltpu.sync_copy(acc_sh.at[pl.ds(sid * ROWS, ROWS)],
                    out_hbm.at[cid, pl.ds(sid * ROWS, ROWS)])


def _prep_body(degp, xp, ew2p, lr, w1l, x2, sl, dinv):
    deg = degp[0, :, 0:1] + degp[1, :, 0:1]
    di = jnp.where(deg > 0, lax.rsqrt(deg), 0.0)
    dinv[...] = di
    x2[...] = xp[...] * di
    t = xp[...] * ew2p[...] * lr[...]
    sl[...] = _gelu(jnp.dot(t, w1l[...], preferred_element_type=jnp.float32))


def _final_body(acc, dinv, sl, w1o, attn, gam, bet, re, wr, out, rel1):
    acc2 = acc[0] + acc[1]
    aggr = jnp.dot(acc2, w1o[...], preferred_element_type=jnp.float32) * dinv[...]
    res1 = _gelu(aggr)
    a = jnp.sum(res1 * attn[...], axis=1, keepdims=True)
    out2 = a * res1
    scale = gam[...] * (1.0 / jnp.sqrt(1.0 + 1e-5))
    full = (0.75 * sl[...] + 0.25 * out2) * scale + bet[...]
    out[...] = full[:NUM_ENT, :]
    rel1[...] = jnp.dot(re[...], wr[...], preferred_element_type=jnp.float32)


def kernel(x, rel_embed, edge_index, edge_type, w1_loop, w1_out, w_rel,
           loop_rel, attn_w, rel_weight1, ent_weight1, ent_weight2,
           bn_gamma, bn_beta):
    ei = edge_index.astype(jnp.int32)
    et = edge_type.astype(jnp.int32)
    npad_e = E_PAD - E_HALF
    pad_dst = (NUM_ENT + (jnp.arange(npad_e) % (NPAD - NUM_ENT))).astype(
        jnp.int32)
    zpad = jnp.zeros((npad_e,), jnp.int32)
    dst = jnp.concatenate([ei[0, E_HALF:], pad_dst])
    src = jnp.concatenate([ei[1, E_HALF:], zpad])
    in0 = jnp.concatenate([ei[0, :E_HALF], zpad])
    typ = jnp.concatenate([et[E_HALF:], zpad])
    idx4 = jnp.stack([
        src.reshape(NW, NCHUNK, CHUNK),
        typ.reshape(NW, NCHUNK, CHUNK),
        in0.reshape(NW, NCHUNK, CHUNK),
        dst.reshape(NW, NCHUNK, CHUNK),
    ], axis=2)
    dstd = dst.reshape(NW, NCHUNK_D, CHUNK_D)

    pad = ((0, NPAD - NUM_ENT), (0, 0))
    xp = jnp.pad(x, pad)
    ew2p = jnp.pad(ent_weight2, pad)

    def _to_i32(a):
        n, m = a.shape
        return lax.bitcast_convert_type(a.reshape(n, m // 2, 2), jnp.int32)

    nrel = rel_embed.shape[0]
    rwt = _to_i32(jnp.concatenate(
        [rel_embed[:, _Q], rel_weight1[:nrel][:, _Q]], axis=1
    ).astype(jnp.bfloat16))

    ones128 = jnp.ones((CHUNK_D, OUT_C), jnp.float32)
    z128 = jnp.zeros((ROWS, OUT_C), jnp.float32)

    degp = _deg_kernel(dstd, ones128, z128)

    x2, sl, dinv = pl.pallas_call(
        _prep_body,
        out_shape=[
            jax.ShapeDtypeStruct((NPAD, IN_C), jnp.float32),
            jax.ShapeDtypeStruct((NPAD, OUT_C), jnp.float32),
            jax.ShapeDtypeStruct((NPAD, 1), jnp.float32),
        ],
    )(degp, xp, ew2p, loop_rel, w1_loop)

    acc = _edge_kernel(x2, rwt, ent_weight1, idx4, z128)

    out, rel1 = pl.pallas_call(
        _final_body,
        out_shape=[
            jax.ShapeDtypeStruct((NUM_ENT, OUT_C), jnp.float32),
            jax.ShapeDtypeStruct((rel_embed.shape[0], OUT_C), jnp.float32),
        ],
    )(acc, dinv, sl, w1_out, attn_w, bn_gamma.reshape(1, OUT_C),
      bn_beta.reshape(1, OUT_C), rel_embed, w_rel)

    return out, rel1
